# baseline (speedup 1.0000x reference)
import functools

import jax
import jax.numpy as jnp
from jax import lax
from jax.experimental import pallas as pl
from jax.experimental.pallas import tpu as pltpu
from jax.experimental.pallas import tpu_sc as plsc

N_POSE_SLOTS = 16
N_TYPES = 20
NC = 2
NS = 16
LANES = 16
RBLK = 4
HI_BITS = 13
POSE_SHIFT = 22


def _prep_body(c_ref, rad_ref, wd_ref, bt_ref, pose_ref, blk_ref, prep_ref):
    c = c_ref[...]
    n = c.shape[0]
    na = c.shape[1] // 3
    ct = c.T
    cen = jnp.mean(ct.reshape(na, 3, n), axis=0)
    prep_ref[0:3, :] = cen

    bt = bt_ref[...]
    sig = jnp.zeros(bt.shape, jnp.float32)
    seps = jnp.zeros(bt.shape, jnp.float32)
    for t in range(N_TYPES):
        sig = jnp.where(bt == t, rad_ref[t], sig)
        seps = jnp.where(bt == t, jnp.sqrt(wd_ref[t]), seps)
    prep_ref[3:4, :] = sig
    prep_ref[4:5, :] = seps

    pose = pose_ref[...]
    hi = jnp.zeros(pose.shape, jnp.int32)
    start = jnp.int32(0)
    for p in range(N_POSE_SLOTS):
        cnt = jnp.sum((pose == p).astype(jnp.int32))
        end = start + cnt
        hi = jnp.where(pose == p, end, hi)
        start = end
    meta = (hi + blk_ref[...] * (1 << HI_BITS)
            + pose * (1 << POSE_SHIFT))
    prep_ref[5:6, :] = lax.bitcast_convert_type(meta, jnp.float32)


def _prep_call(coords2, lj_radius, lj_wdepth, bt2, pose2, blk2):
    n = coords2.shape[0]
    return pl.pallas_call(
        _prep_body,
        out_shape=jax.ShapeDtypeStruct((6, n), jnp.float32),
        in_specs=[
            pl.BlockSpec(memory_space=pltpu.VMEM),
            pl.BlockSpec(memory_space=pltpu.SMEM),
            pl.BlockSpec(memory_space=pltpu.SMEM),
            pl.BlockSpec(memory_space=pltpu.VMEM),
            pl.BlockSpec(memory_space=pltpu.VMEM),
            pl.BlockSpec(memory_space=pltpu.VMEM),
        ],
    )(coords2, lj_radius, lj_wdepth, bt2, pose2, blk2)


def _sc_body(n_rots, prep_hbm, out_hbm, pv, sharedv, stagev, accv, dsem):
    wid = lax.axis_index("s") * NC + lax.axis_index("c")
    sid = lax.axis_index("s")
    n_blocks = n_rots // (RBLK * NC * NS)
    stride = RBLK * NC * NS
    cols = n_rots // NS

    seg = pl.ds(sid * cols, cols)
    pltpu.sync_copy(prep_hbm.at[:, seg], stagev)
    pltpu.sync_copy(stagev, sharedv.at[:, seg])

    lane_iota = lax.iota(jnp.int32, LANES)
    zeros = jnp.zeros((LANES,), jnp.float32)
    zidx = jnp.zeros((LANES,), jnp.int32)
    for q in range(N_POSE_SLOTS):
        accv[pl.ds(q * LANES, LANES)] = zeros

    plsc.subcore_barrier()
    pltpu.sync_copy(sharedv, pv)

    def blk_body(k, carry):
        i0 = wid * RBLK + k * stride
        last = i0 + (RBLK - 1)
        g0 = jnp.bitwise_and(last, jnp.int32(-LANES))
        hvec = jnp.bitwise_and(
            plsc.bitcast(pv[5, pl.ds(g0, LANES)], jnp.int32),
            jnp.int32((1 << HI_BITS) - 1))
        hi_max = jnp.sum(jnp.where(lane_iota == (last - g0), hvec,
                                   jnp.zeros_like(hvec)))

        rows = []
        for r in range(RBLK):
            isplat = jnp.full((LANES,), i0 + r, jnp.int32)
            mi = plsc.bitcast(plsc.load_gather(pv, [zidx + 5, isplat]),
                              jnp.int32)
            rows.append((
                plsc.load_gather(pv, [zidx, isplat]),
                plsc.load_gather(pv, [zidx + 1, isplat]),
                plsc.load_gather(pv, [zidx + 2, isplat]),
                plsc.load_gather(pv, [zidx + 3, isplat]),
                plsc.load_gather(pv, [zidx + 4, isplat]),
                lax.shift_right_logical(mi, HI_BITS),
                jnp.bitwise_and(mi, jnp.int32((1 << HI_BITS) - 1)),
                lax.shift_right_logical(mi, POSE_SHIFT),
            ))

        jstart = jnp.bitwise_and(i0 + 1, jnp.int32(-LANES))
        n_it = lax.shift_right_arithmetic(hi_max - jstart + (LANES - 1), 4)
        t_peel = lax.shift_right_arithmetic(
            i0 + (RBLK + LANES - 1) - jstart, 4)

        def make_col_body(triangular):
            def col_body(t, accs):
                js = jstart + t * LANES
                jvec = js + lane_iota
                xj = pv[0, pl.ds(js, LANES)]
                yj = pv[1, pl.ds(js, LANES)]
                zj = pv[2, pl.ds(js, LANES)]
                sj = pv[3, pl.ds(js, LANES)]
                ej = pv[4, pl.ds(js, LANES)]
                mj = plsc.bitcast(pv[5, pl.ds(js, LANES)], jnp.int32)
                kj = lax.shift_right_logical(mj, HI_BITS)
                out = []
                for r in range(RBLK):
                    xi, yi, zi, si, ei, ki, hi_i, _ = rows[r]
                    dx = xi - xj
                    dy = yi - yj
                    dz = zi - zj
                    d2 = jnp.maximum(dx * dx + dy * dy + dz * dz,
                                     jnp.float32(0.01))
                    s = si + sj
                    q = (s * s) / d2
                    q3 = q * q * q
                    t6 = ej * (q3 * (q3 - 2.0))
                    m = (jvec < hi_i) & (kj != ki)
                    if triangular:
                        m = m & (jvec > (i0 + r))
                    out.append(accs[r] + jnp.where(m, t6, jnp.float32(0.0)))
                return tuple(out)
            return col_body

        accs = lax.fori_loop(0, t_peel, make_col_body(True),
                             tuple(zeros for _ in range(RBLK)))
        accs = lax.fori_loop(t_peel, n_it, make_col_body(False), accs)

        for r in range(RBLK):
            _, _, _, _, ei, _, _, pi = rows[r]
            idx = pi * LANES + lane_iota
            plsc.addupdate_scatter(accv, [idx], ei * accs[r])
        return carry

    lax.fori_loop(0, n_blocks, blk_body, jnp.int32(0))
    pltpu.sync_copy(accv, out_hbm.at[pl.ds(wid * (N_POSE_SLOTS * LANES),
                                           N_POSE_SLOTS * LANES)])


def _sc_call(prep):
    n = prep.shape[-1]
    nw = NC * NS
    mesh = plsc.VectorSubcoreMesh(core_axis_name="c", subcore_axis_name="s",
                                  num_cores=NC, num_subcores=NS)
    kern = functools.partial(
        pl.kernel,
        out_type=jax.ShapeDtypeStruct((nw * N_POSE_SLOTS * LANES,),
                                      jnp.float32),
        mesh=mesh,
        compiler_params=pltpu.CompilerParams(needs_layout_passes=False),
        scratch_types=[
            pltpu.VMEM((6, n), jnp.float32),
            pltpu.VMEM_SHARED((6, n), jnp.float32),
            pltpu.VMEM((6, n // NS), jnp.float32),
            pltpu.VMEM((N_POSE_SLOTS * LANES,), jnp.float32),
            pltpu.SemaphoreType.DMA,
        ],
    )(functools.partial(_sc_body, n))
    return kern(prep)


def kernel(coords, lj_radius, lj_wdepth, pose_ind_for_rot, block_ind_for_rot,
           block_type_ind_for_rot):
    n = coords.shape[0]
    coords2 = coords.reshape(n, -1)
    bt2 = block_type_ind_for_rot.reshape(1, n)
    pose2 = pose_ind_for_rot.reshape(1, n)
    blk2 = block_ind_for_rot.reshape(1, n)
    prep = _prep_call(coords2, lj_radius, lj_wdepth, bt2, pose2, blk2)
    partials = _sc_call(prep)
    return jnp.sum(partials.reshape(NC * NS, N_POSE_SLOTS, LANES),
                   axis=(0, 2))


# speedup vs baseline: 1.1594x; 1.1594x over previous
import functools

import jax
import jax.numpy as jnp
from jax import lax
from jax.experimental import pallas as pl
from jax.experimental.pallas import tpu as pltpu
from jax.experimental.pallas import tpu_sc as plsc

N_POSE_SLOTS = 16
N_TYPES = 20
NC = 2
NS = 16
LANES = 16
RBLK = 4
HI_BITS = 13
POSE_SHIFT = 22


def _prep_body(c_ref, rad_ref, wd_ref, bt_ref, pose_ref, blk_ref, prep_ref):
    c = c_ref[...]
    cen = jnp.mean(c, axis=1)
    prep_ref[0:3, :] = cen

    bt = bt_ref[...]
    sig = jnp.zeros(bt.shape, jnp.float32)
    seps = jnp.zeros(bt.shape, jnp.float32)
    for t in range(N_TYPES):
        sig = jnp.where(bt == t, rad_ref[t], sig)
        seps = jnp.where(bt == t, jnp.sqrt(wd_ref[t]), seps)
    prep_ref[3:4, :] = sig
    prep_ref[4:5, :] = seps

    pose = pose_ref[...]
    hi = jnp.zeros(pose.shape, jnp.int32)
    start = jnp.int32(0)
    for p in range(N_POSE_SLOTS):
        cnt = jnp.sum((pose == p).astype(jnp.int32))
        end = start + cnt
        hi = jnp.where(pose == p, end, hi)
        start = end
    meta = (hi + blk_ref[...] * (1 << HI_BITS)
            + pose * (1 << POSE_SHIFT))
    prep_ref[5:6, :] = lax.bitcast_convert_type(meta, jnp.float32)


def _prep_call(coords3, lj_radius, lj_wdepth, bt2, pose2, blk2):
    n = coords3.shape[-1]
    return pl.pallas_call(
        _prep_body,
        out_shape=jax.ShapeDtypeStruct((6, n), jnp.float32),
        in_specs=[
            pl.BlockSpec(memory_space=pltpu.VMEM),
            pl.BlockSpec(memory_space=pltpu.SMEM),
            pl.BlockSpec(memory_space=pltpu.SMEM),
            pl.BlockSpec(memory_space=pltpu.VMEM),
            pl.BlockSpec(memory_space=pltpu.VMEM),
            pl.BlockSpec(memory_space=pltpu.VMEM),
        ],
    )(coords3, lj_radius, lj_wdepth, bt2, pose2, blk2)


def _sc_body(n_rots, prep_hbm, out_hbm, pv, sharedv, stagev, accv, dsem):
    wid = lax.axis_index("s") * NC + lax.axis_index("c")
    sid = lax.axis_index("s")
    n_blocks = n_rots // (RBLK * NC * NS)
    stride = RBLK * NC * NS
    cols = n_rots // NS

    seg = pl.ds(sid * cols, cols)
    pltpu.sync_copy(prep_hbm.at[:, seg], stagev)
    pltpu.sync_copy(stagev, sharedv.at[:, seg])

    lane_iota = lax.iota(jnp.int32, LANES)
    zeros = jnp.zeros((LANES,), jnp.float32)
    zidx = jnp.zeros((LANES,), jnp.int32)
    for q in range(N_POSE_SLOTS):
        accv[pl.ds(q * LANES, LANES)] = zeros

    plsc.subcore_barrier()
    pltpu.sync_copy(sharedv, pv)

    def blk_body(k, carry):
        i0 = wid * RBLK + k * stride
        last = i0 + (RBLK - 1)
        g0 = jnp.bitwise_and(last, jnp.int32(-LANES))
        hvec = jnp.bitwise_and(
            plsc.bitcast(pv[5, pl.ds(g0, LANES)], jnp.int32),
            jnp.int32((1 << HI_BITS) - 1))
        hi_max = jnp.sum(jnp.where(lane_iota == (last - g0), hvec,
                                   jnp.zeros_like(hvec)))

        rows = []
        for r in range(RBLK):
            isplat = jnp.full((LANES,), i0 + r, jnp.int32)
            mi = plsc.bitcast(plsc.load_gather(pv, [zidx + 5, isplat]),
                              jnp.int32)
            rows.append((
                plsc.load_gather(pv, [zidx, isplat]),
                plsc.load_gather(pv, [zidx + 1, isplat]),
                plsc.load_gather(pv, [zidx + 2, isplat]),
                plsc.load_gather(pv, [zidx + 3, isplat]),
                plsc.load_gather(pv, [zidx + 4, isplat]),
                lax.shift_right_logical(mi, HI_BITS),
                jnp.bitwise_and(mi, jnp.int32((1 << HI_BITS) - 1)),
                lax.shift_right_logical(mi, POSE_SHIFT),
            ))

        jstart = jnp.bitwise_and(i0 + 1, jnp.int32(-LANES))
        n_it = lax.shift_right_arithmetic(hi_max - jstart + (LANES - 1), 4)
        t_peel = lax.shift_right_arithmetic(
            i0 + (RBLK + LANES - 1) - jstart, 4)

        def make_col_body(triangular):
            def col_body(t, accs):
                js = jstart + t * LANES
                jvec = js + lane_iota
                xj = pv[0, pl.ds(js, LANES)]
                yj = pv[1, pl.ds(js, LANES)]
                zj = pv[2, pl.ds(js, LANES)]
                sj = pv[3, pl.ds(js, LANES)]
                ej = pv[4, pl.ds(js, LANES)]
                mj = plsc.bitcast(pv[5, pl.ds(js, LANES)], jnp.int32)
                kj = lax.shift_right_logical(mj, HI_BITS)
                out = []
                for r in range(RBLK):
                    xi, yi, zi, si, ei, ki, hi_i, _ = rows[r]
                    dx = xi - xj
                    dy = yi - yj
                    dz = zi - zj
                    d2 = jnp.maximum(dx * dx + dy * dy + dz * dz,
                                     jnp.float32(0.01))
                    s = si + sj
                    q = (s * s) / d2
                    q3 = q * q * q
                    t6 = ej * (q3 * (q3 - 2.0))
                    m = (jvec < hi_i) & (kj != ki)
                    if triangular:
                        m = m & (jvec > (i0 + r))
                    out.append(accs[r] + jnp.where(m, t6, jnp.float32(0.0)))
                return tuple(out)
            return col_body

        accs = lax.fori_loop(0, t_peel, make_col_body(True),
                             tuple(zeros for _ in range(RBLK)))
        accs = lax.fori_loop(t_peel, n_it, make_col_body(False), accs)

        for r in range(RBLK):
            _, _, _, _, ei, _, _, pi = rows[r]
            idx = pi * LANES + lane_iota
            plsc.addupdate_scatter(accv, [idx], ei * accs[r])
        return carry

    lax.fori_loop(0, n_blocks, blk_body, jnp.int32(0))
    pltpu.sync_copy(accv, out_hbm.at[pl.ds(wid * (N_POSE_SLOTS * LANES),
                                           N_POSE_SLOTS * LANES)])


def _sc_call(prep):
    n = prep.shape[-1]
    nw = NC * NS
    mesh = plsc.VectorSubcoreMesh(core_axis_name="c", subcore_axis_name="s",
                                  num_cores=NC, num_subcores=NS)
    kern = functools.partial(
        pl.kernel,
        out_type=jax.ShapeDtypeStruct((nw * N_POSE_SLOTS * LANES,),
                                      jnp.float32),
        mesh=mesh,
        compiler_params=pltpu.CompilerParams(needs_layout_passes=False),
        scratch_types=[
            pltpu.VMEM((6, n), jnp.float32),
            pltpu.VMEM_SHARED((6, n), jnp.float32),
            pltpu.VMEM((6, n // NS), jnp.float32),
            pltpu.VMEM((N_POSE_SLOTS * LANES,), jnp.float32),
            pltpu.SemaphoreType.DMA,
        ],
    )(functools.partial(_sc_body, n))
    return kern(prep)


def kernel(coords, lj_radius, lj_wdepth, pose_ind_for_rot, block_ind_for_rot,
           block_type_ind_for_rot):
    n = coords.shape[0]
    coords3 = coords.transpose(2, 1, 0)
    bt2 = block_type_ind_for_rot.reshape(1, n)
    pose2 = pose_ind_for_rot.reshape(1, n)
    blk2 = block_ind_for_rot.reshape(1, n)
    prep = _prep_call(coords3, lj_radius, lj_wdepth, bt2, pose2, blk2)
    partials = _sc_call(prep)
    return jnp.sum(partials.reshape(NC * NS, N_POSE_SLOTS, LANES),
                   axis=(0, 2))
